# baseline (speedup 1.0000x reference)
import functools

import jax
import jax.numpy as jnp
from jax import lax
from jax.experimental import pallas as pl
from jax.experimental.pallas import tpu as pltpu
from jax.experimental.pallas import tpu_sc as plsc

R, C = 64, 32768
B = 4096
L = 16
NCHUNK = 4
CB = B // NCHUNK

_info = plsc.get_sparse_core_info()
_NC, _NS = _info.num_cores, _info.num_subcores
NW = _NC * _NS
ROWS_PER_W = R // NW

_mesh = plsc.VectorSubcoreMesh(core_axis_name="c", subcore_axis_name="s")


@functools.partial(
    pl.kernel,
    mesh=_mesh,
    out_type=jax.ShapeDtypeStruct((R, B), jnp.float32),
    scratch_types=[
        [pltpu.VMEM((C,), jnp.float32) for _ in range(ROWS_PER_W)],
        [pltpu.VMEM((B,), jnp.int32) for _ in range(ROWS_PER_W)],
        [pltpu.VMEM((B,), jnp.float32) for _ in range(ROWS_PER_W)],
        [pltpu.SemaphoreType.DMA for _ in range(ROWS_PER_W)],
        [pltpu.SemaphoreType.DMA for _ in range(ROWS_PER_W)],
    ],
    compiler_params=pltpu.CompilerParams(
        needs_layout_passes=False,
    ),
)
def _gather_rows(x_hbm, y_hbm, out_hbm, rows_v, idxs_v, outs_v, in_sems, out_sems):
    wid = lax.axis_index("s") * _NC + lax.axis_index("c")

    in_copies = []
    for r in range(ROWS_PER_W):
        row = wid * ROWS_PER_W + r
        cx = pltpu.make_async_copy(x_hbm.at[row], rows_v[r], in_sems[r])
        cy = pltpu.make_async_copy(y_hbm.at[row], idxs_v[r], in_sems[r])
        cx.start()
        cy.start()
        in_copies.append((cx, cy))

    out_copies = []
    for r in range(ROWS_PER_W):
        row = wid * ROWS_PER_W + r
        cx, cy = in_copies[r]
        cx.wait()
        cy.wait()
        row_v, idx_v, out_v = rows_v[r], idxs_v[r], outs_v[r]

        for c in range(NCHUNK):

            @plsc.parallel_loop(c * (CB // L), (c + 1) * (CB // L), unroll=8)
            def _(j):
                base = j * L
                idx = idx_v[pl.ds(base, L)]
                out_v[pl.ds(base, L)] = plsc.load_gather(row_v, [idx])

            co = pltpu.make_async_copy(
                out_v.at[pl.ds(c * CB, CB)],
                out_hbm.at[row, pl.ds(c * CB, CB)],
                out_sems[r],
            )
            co.start()
            out_copies.append(co)

    for co in out_copies:
        co.wait()


def kernel(x, y):
    return _gather_rows(x, y.astype(jnp.int32))


# speedup vs baseline: 1.0211x; 1.0211x over previous
import functools

import jax
import jax.numpy as jnp
from jax import lax
from jax.experimental import pallas as pl
from jax.experimental.pallas import tpu as pltpu
from jax.experimental.pallas import tpu_sc as plsc

R, C = 64, 32768
B = 4096
L = 16

_info = plsc.get_sparse_core_info()
_NC, _NS = _info.num_cores, _info.num_subcores
NW = _NC * _NS
ROWS_PER_W = R // NW

_mesh = plsc.VectorSubcoreMesh(core_axis_name="c", subcore_axis_name="s")


@functools.partial(
    pl.kernel,
    mesh=_mesh,
    out_type=jax.ShapeDtypeStruct((R, B), jnp.float32),
    scratch_types=[
        [pltpu.VMEM((C,), jnp.float32) for _ in range(ROWS_PER_W)],
        [pltpu.VMEM((B,), jnp.int32) for _ in range(ROWS_PER_W)],
        [pltpu.VMEM((B,), jnp.float32) for _ in range(ROWS_PER_W)],
        [pltpu.SemaphoreType.DMA for _ in range(ROWS_PER_W)],
        [pltpu.SemaphoreType.DMA for _ in range(ROWS_PER_W)],
    ],
    compiler_params=pltpu.CompilerParams(
        needs_layout_passes=False,
    ),
)
def _gather_rows(x_hbm, y_hbm, out_hbm, rows_v, idxs_v, outs_v, in_sems, out_sems):
    wid = lax.axis_index("s") * _NC + lax.axis_index("c")

    in_copies = []
    for r in range(ROWS_PER_W):
        row = wid * ROWS_PER_W + r
        cx = pltpu.make_async_copy(x_hbm.at[row], rows_v[r], in_sems[r])
        cy = pltpu.make_async_copy(y_hbm.at[row], idxs_v[r], in_sems[r])
        cx.start()
        cy.start()
        in_copies.append((cx, cy))

    out_copies = []
    for r in range(ROWS_PER_W):
        row = wid * ROWS_PER_W + r
        cx, cy = in_copies[r]
        cx.wait()
        cy.wait()
        row_v, idx_v, out_v = rows_v[r], idxs_v[r], outs_v[r]

        @plsc.parallel_loop(0, B // L, unroll=8)
        def _(j):
            base = j * L
            idx = idx_v[pl.ds(base, L)]
            out_v[pl.ds(base, L)] = plsc.load_gather(row_v, [idx])

        co = pltpu.make_async_copy(out_v, out_hbm.at[row], out_sems[r])
        co.start()
        out_copies.append(co)

    for co in out_copies:
        co.wait()


def kernel(x, y):
    return _gather_rows(x, y.astype(jnp.int32))
